# Initial kernel scaffold; baseline (speedup 1.0000x reference)
import jax
import jax.numpy as jnp
from jax.experimental import pallas as pl


def kernel(reg_x, sum_x, Wq, Wk, Wv, Wo, reg_ln_g, reg_ln_b, sum_ln_g, sum_ln_b, reg_fln_g, reg_fln_b, sum_fln_g, sum_fln_b, reg_fc1_w, reg_fc1_b, reg_fc2_w, reg_fc2_b, sum_fc1_w, sum_fc1_b, sum_fc2_w, sum_fc2_b):
    raise NotImplementedError("write your pallas kernel here")


# speedup vs baseline: 2.7073x; 2.7073x over previous
import functools

import jax
import jax.numpy as jnp
from jax.experimental import pallas as pl
from jax.experimental.pallas import tpu as pltpu

EMBED_DIM = 768
FFN_DIM = 3072
NUM_HEADS = 12
HEAD_DIM = EMBED_DIM // NUM_HEADS
CHUNK_LEN = 256
REG_LEN = 2048
NUM_CHUNKS = REG_LEN // CHUNK_LEN
SUM_LEN = NUM_CHUNKS
SCALE = 1.0 / (HEAD_DIM ** 0.5)
NEG = -1e9


def _ln(x, g, b):
    m = jnp.mean(x, axis=-1, keepdims=True)
    v = jnp.mean((x - m) ** 2, axis=-1, keepdims=True)
    return (x - m) * jax.lax.rsqrt(v + 1e-5) * g + b


def _dot(a, b):
    return jnp.dot(a, b, preferred_element_type=jnp.float32)


def _dot_t(a, b):
    return jax.lax.dot_general(a, b, (((1,), (1,)), ((), ())),
                               preferred_element_type=jnp.float32)


def _body(reg_x_ref, sum_x_ref, wq_ref, wk_ref, wv_ref, wo_ref,
          reg_ln_g_ref, reg_ln_b_ref, sum_ln_g_ref, sum_ln_b_ref,
          reg_fln_g_ref, reg_fln_b_ref, sum_fln_g_ref, sum_fln_b_ref,
          rfc1w_ref, rfc1b_ref, rfc2w_ref, rfc2b_ref,
          sfc1w_ref, sfc1b_ref, sfc2w_ref, sfc2b_ref,
          out_reg_ref, out_sum_ref,
          qs_ref, ks_ref, vs_ref, ssc_ref, vall_ref):
    c = pl.program_id(0)

    @pl.when(c == 0)
    def _init_summary_qkv():
        hs = _ln(sum_x_ref[...], sum_ln_g_ref[...], sum_ln_b_ref[...])
        qs_ref[...] = _dot(hs, wq_ref[...])
        ks_ref[...] = _dot(hs, wk_ref[...])
        vs_ref[...] = _dot(hs, wv_ref[...])

    x0 = reg_x_ref[...]
    h = _ln(x0, reg_ln_g_ref[...], reg_ln_b_ref[...])
    q = _dot(h, wq_ref[...])
    k = _dot(h, wk_ref[...])
    v = _dot(h, wv_ref[...])
    vall_ref[pl.ds(c * CHUNK_LEN, CHUNK_LEN), :] = v

    q_sum = qs_ref[...]
    k_sum = ks_ref[...]
    v_sum = vs_ref[...]

    row = jax.lax.broadcasted_iota(jnp.int32, (CHUNK_LEN, CHUNK_LEN), 0)
    col = jax.lax.broadcasted_iota(jnp.int32, (CHUNK_LEN, CHUNK_LEN), 1)
    causal = row >= col
    col_s = jax.lax.broadcasted_iota(jnp.int32, (CHUNK_LEN, SUM_LEN), 1)
    sum_key_ok = col_s < c

    ctxs = []
    for hd in range(NUM_HEADS):
        sl = slice(hd * HEAD_DIM, (hd + 1) * HEAD_DIM)
        qh, kh, vh = q[:, sl], k[:, sl], v[:, sl]
        s_loc = jnp.where(causal, _dot_t(qh, kh) * SCALE, NEG)
        s_sm = jnp.where(sum_key_ok, _dot_t(qh, k_sum[:, sl]) * SCALE, NEG)
        m = jnp.maximum(jnp.max(s_loc, axis=-1, keepdims=True),
                        jnp.max(s_sm, axis=-1, keepdims=True))
        e_loc = jnp.exp(s_loc - m)
        e_sm = jnp.exp(s_sm - m)
        l = (jnp.sum(e_loc, axis=-1, keepdims=True)
             + jnp.sum(e_sm, axis=-1, keepdims=True))
        ctxs.append((_dot(e_loc, vh) + _dot(e_sm, v_sum[:, sl])) / l)
        ssc_ref[hd, :, pl.ds(c * CHUNK_LEN, CHUNK_LEN)] = (
            _dot_t(q_sum[:, sl], kh) * SCALE)

    ctx = jnp.concatenate(ctxs, axis=1)
    x = x0 + _dot(ctx, wo_ref[...])
    f = _ln(x, reg_fln_g_ref[...], reg_fln_b_ref[...])
    ffn = jnp.maximum(_dot(f, rfc1w_ref[...]) + rfc1b_ref[...], 0.0)
    out_reg_ref[...] = x + _dot(ffn, rfc2w_ref[...]) + rfc2b_ref[...]

    @pl.when(c == NUM_CHUNKS - 1)
    def _finalize_summary():
        row8 = jax.lax.broadcasted_iota(jnp.int32, (SUM_LEN, SUM_LEN), 0)
        col8 = jax.lax.broadcasted_iota(jnp.int32, (SUM_LEN, SUM_LEN), 1)
        ss_ok = col8 <= row8
        rowr = jax.lax.broadcasted_iota(jnp.int32, (SUM_LEN, REG_LEN), 0)
        colr = jax.lax.broadcasted_iota(jnp.int32, (SUM_LEN, REG_LEN), 1)
        sr_ok = (colr // CHUNK_LEN) <= rowr
        ctxs_s = []
        for hd in range(NUM_HEADS):
            sl = slice(hd * HEAD_DIM, (hd + 1) * HEAD_DIM)
            s_ss = jnp.where(ss_ok, _dot_t(q_sum[:, sl], k_sum[:, sl]) * SCALE,
                             NEG)
            s_sr = jnp.where(sr_ok, ssc_ref[hd], NEG)
            m = jnp.maximum(jnp.max(s_ss, axis=-1, keepdims=True),
                            jnp.max(s_sr, axis=-1, keepdims=True))
            e_ss = jnp.exp(s_ss - m)
            e_sr = jnp.exp(s_sr - m)
            l = (jnp.sum(e_ss, axis=-1, keepdims=True)
                 + jnp.sum(e_sr, axis=-1, keepdims=True))
            ctxs_s.append((_dot(e_ss, v_sum[:, sl])
                           + _dot(e_sr, vall_ref[:, sl])) / l)
        ctx_s = jnp.concatenate(ctxs_s, axis=1)
        xs = sum_x_ref[...] + _dot(ctx_s, wo_ref[...])
        fs = _ln(xs, sum_fln_g_ref[...], sum_fln_b_ref[...])
        ffn_s = jnp.maximum(_dot(fs, sfc1w_ref[...]) + sfc1b_ref[...], 0.0)
        out_sum_ref[...] = xs + _dot(ffn_s, sfc2w_ref[...]) + sfc2b_ref[...]


@functools.partial(jax.jit, static_argnames=("interpret",))
def _run(reg_x, sum_x, Wq, Wk, Wv, Wo, reg_ln_g, reg_ln_b, sum_ln_g, sum_ln_b,
         reg_fln_g, reg_fln_b, sum_fln_g, sum_fln_b,
         reg_fc1_w, reg_fc1_b, reg_fc2_w, reg_fc2_b,
         sum_fc1_w, sum_fc1_b, sum_fc2_w, sum_fc2_b, interpret=False):
    full = lambda shape: pl.BlockSpec(shape, lambda c: (0,) * len(shape))
    in_specs = [
        pl.BlockSpec((CHUNK_LEN, EMBED_DIM), lambda c: (c, 0)),
        full((SUM_LEN, EMBED_DIM)),
        full((EMBED_DIM, EMBED_DIM)),
        full((EMBED_DIM, EMBED_DIM)),
        full((EMBED_DIM, EMBED_DIM)),
        full((EMBED_DIM, EMBED_DIM)),
        full((1, EMBED_DIM)), full((1, EMBED_DIM)),
        full((1, EMBED_DIM)), full((1, EMBED_DIM)),
        full((1, EMBED_DIM)), full((1, EMBED_DIM)),
        full((1, EMBED_DIM)), full((1, EMBED_DIM)),
        full((EMBED_DIM, FFN_DIM)), full((1, FFN_DIM)),
        full((FFN_DIM, EMBED_DIM)), full((1, EMBED_DIM)),
        full((EMBED_DIM, FFN_DIM)), full((1, FFN_DIM)),
        full((FFN_DIM, EMBED_DIM)), full((1, EMBED_DIM)),
    ]
    out_specs = [
        pl.BlockSpec((CHUNK_LEN, EMBED_DIM), lambda c: (c, 0)),
        full((SUM_LEN, EMBED_DIM)),
    ]
    out_reg, out_sum = pl.pallas_call(
        _body,
        grid=(NUM_CHUNKS,),
        in_specs=in_specs,
        out_specs=out_specs,
        out_shape=[
            jax.ShapeDtypeStruct((REG_LEN, EMBED_DIM), jnp.float32),
            jax.ShapeDtypeStruct((SUM_LEN, EMBED_DIM), jnp.float32),
        ],
        scratch_shapes=[
            pltpu.VMEM((SUM_LEN, EMBED_DIM), jnp.float32),
            pltpu.VMEM((SUM_LEN, EMBED_DIM), jnp.float32),
            pltpu.VMEM((SUM_LEN, EMBED_DIM), jnp.float32),
            pltpu.VMEM((NUM_HEADS, SUM_LEN, REG_LEN), jnp.float32),
            pltpu.VMEM((REG_LEN, EMBED_DIM), jnp.float32),
        ],
        compiler_params=pltpu.CompilerParams(
            vmem_limit_bytes=100 * 1024 * 1024),
        interpret=interpret,
    )(
        reg_x[0], sum_x[0], Wq, Wk, Wv, Wo,
        reg_ln_g[None], reg_ln_b[None], sum_ln_g[None], sum_ln_b[None],
        reg_fln_g[None], reg_fln_b[None], sum_fln_g[None], sum_fln_b[None],
        reg_fc1_w, reg_fc1_b[None], reg_fc2_w, reg_fc2_b[None],
        sum_fc1_w, sum_fc1_b[None], sum_fc2_w, sum_fc2_b[None],
    )
    return jnp.concatenate([out_sum, out_reg], axis=0)[None]


def kernel(reg_x, sum_x, Wq, Wk, Wv, Wo, reg_ln_g, reg_ln_b, sum_ln_g,
           sum_ln_b, reg_fln_g, reg_fln_b, sum_fln_g, sum_fln_b,
           reg_fc1_w, reg_fc1_b, reg_fc2_w, reg_fc2_b,
           sum_fc1_w, sum_fc1_b, sum_fc2_w, sum_fc2_b):
    return _run(reg_x, sum_x, Wq, Wk, Wv, Wo, reg_ln_g, reg_ln_b, sum_ln_g,
                sum_ln_b, reg_fln_g, reg_fln_b, sum_fln_g, sum_fln_b,
                reg_fc1_w, reg_fc1_b, reg_fc2_w, reg_fc2_b,
                sum_fc1_w, sum_fc1_b, sum_fc2_w, sum_fc2_b)
